# baseline (speedup 1.0000x reference)
import jax
import jax.numpy as jnp
from jax.experimental import pallas as pl
from jax.experimental.pallas import tpu as pltpu

_NB = 10


def _fused_mlp_kernel(x_ref, w1_ref, gamma_ref, beta_ref,
                      w2_ref, b2_ref, out_ref, x_sc, g_sc, s_sc):
    i = pl.program_id(0)
    rows = x_ref.shape[0]
    n_total = rows * _NB

    @pl.when(i < _NB)
    def _stream_and_accumulate():
        xb = x_ref[...]
        x_sc[pl.ds(i * rows, rows), :] = xb
        gb = jax.lax.dot_general(
            xb, xb, (((0,), (0,)), ((), ())),
            preferred_element_type=jnp.float32,
        )
        sb = jnp.sum(xb, axis=0, keepdims=True)

        @pl.when(i == 0)
        def _():
            g_sc[...] = gb
            s_sc[0:1, :] = sb

        @pl.when(i > 0)
        def _():
            g_sc[...] = g_sc[...] + gb
            s_sc[0:1, :] = s_sc[0:1, :] + sb

    @pl.when(i == _NB)
    def _fold_and_mlp():
        inv_n = 1.0 / n_total
        w1 = w1_ref[...]
        m = s_sc[0:1, :] * inv_n
        u = jax.lax.dot_general(
            w1, m, (((1,), (1,)), ((), ())),
            preferred_element_type=jnp.float32,
        )
        a = jax.lax.dot_general(
            w1, g_sc[...] * inv_n, (((1,), (0,)), ((), ())),
            preferred_element_type=jnp.float32,
        )
        e2 = jnp.sum(a * w1, axis=1, keepdims=True)
        var = e2 - u * u
        scale = gamma_ref[...][:, None] * jax.lax.rsqrt(var + 1e-5)
        w1f = w1 * scale
        bias = beta_ref[...] - (u * scale)[:, 0]

        h = jax.lax.dot_general(
            x_sc[...], w1f, (((1,), (1,)), ((), ())),
            preferred_element_type=jnp.float32,
        ) + bias
        h = jnp.maximum(h, 0.0)

        logits_t = jax.lax.dot_general(
            w2_ref[...], h, (((1,), (1,)), ((), ())),
            preferred_element_type=jnp.float32,
        ) + b2_ref[...][:, None]

        mx = jnp.max(logits_t, axis=0, keepdims=True)
        shifted = logits_t - mx
        lse = jnp.log(jnp.sum(jnp.exp(shifted), axis=0, keepdims=True))
        out_ref[...] = shifted - lse


def kernel(x, edge_index, W1, b1, gamma, beta, W2, b2):
    del edge_index
    del b1
    n, feat = x.shape
    hid = W1.shape[0]
    nclass = W2.shape[0]
    rows = n // _NB

    out_t = pl.pallas_call(
        _fused_mlp_kernel,
        grid=(_NB + 1,),
        in_specs=[
            pl.BlockSpec((rows, feat), lambda i: (jnp.minimum(i, _NB - 1), 0)),
            pl.BlockSpec((hid, feat), lambda i: (0, 0)),
            pl.BlockSpec((hid,), lambda i: (0,)),
            pl.BlockSpec((hid,), lambda i: (0,)),
            pl.BlockSpec((nclass, hid), lambda i: (0, 0)),
            pl.BlockSpec((nclass,), lambda i: (0,)),
        ],
        out_specs=pl.BlockSpec((nclass, n), lambda i: (0, 0)),
        out_shape=jax.ShapeDtypeStruct((nclass, n), jnp.float32),
        scratch_shapes=[
            pltpu.VMEM((n, feat), jnp.float32),
            pltpu.VMEM((feat, feat), jnp.float32),
            pltpu.VMEM((8, feat), jnp.float32),
        ],
        compiler_params=pltpu.CompilerParams(
            dimension_semantics=("arbitrary",),
        ),
    )(x, W1, gamma, beta, W2, b2)
    return out_t.T


# speedup vs baseline: 1.2559x; 1.2559x over previous
import jax
import jax.numpy as jnp
from jax.experimental import pallas as pl
from jax.experimental.pallas import tpu as pltpu

_NCHUNK = 5


def _fused_mlp_kernel(x_hbm, w1_ref, gamma_ref, beta_ref, w2_ref, b2_ref,
                      out_ref, xbuf, h_sc, st_sc, sems):
    n, feat = h_sc.shape
    rows = n // _NCHUNK
    w1 = w1_ref[...]

    def _copy(k, slot):
        return pltpu.make_async_copy(
            x_hbm.at[pl.ds(k * rows, rows), :], xbuf.at[slot], sems.at[slot])

    _copy(0, 0).start()

    def _body(k, carry):
        s, q = carry
        slot = jax.lax.rem(k, 2)

        @pl.when(k + 1 < _NCHUNK)
        def _():
            _copy(k + 1, jax.lax.rem(k + 1, 2)).start()

        _copy(k, slot).wait()
        xb = xbuf[slot]
        hb = jax.lax.dot_general(
            xb, w1, (((1,), (1,)), ((), ())),
            preferred_element_type=jnp.float32,
        )
        h_sc[pl.ds(k * rows, rows), :] = hb
        s = s + jnp.sum(hb, axis=0, keepdims=True)
        q = q + jnp.sum(hb * hb, axis=0, keepdims=True)
        return s, q

    zero = jnp.zeros((1, feat), dtype=jnp.float32)
    s, q = jax.lax.fori_loop(0, _NCHUNK, _body, (zero, zero))

    inv_n = 1.0 / n
    mean = s * inv_n
    var = q * inv_n - mean * mean
    scale = gamma_ref[...][None, :] * jax.lax.rsqrt(var + 1e-5)
    shift = beta_ref[...][None, :] - mean * scale

    hn = jnp.maximum(h_sc[...] * scale + shift, 0.0)
    logits_t = jax.lax.dot_general(
        w2_ref[...], hn, (((1,), (1,)), ((), ())),
        preferred_element_type=jnp.float32,
    ) + b2_ref[...][:, None]

    mx = jnp.max(logits_t, axis=0, keepdims=True)
    shifted = logits_t - mx
    lse = jnp.log(jnp.sum(jnp.exp(shifted), axis=0, keepdims=True))
    out_ref[...] = shifted - lse


def kernel(x, edge_index, W1, b1, gamma, beta, W2, b2):
    del edge_index
    del b1
    n, feat = x.shape
    hid = W1.shape[0]
    nclass = W2.shape[0]
    rows = n // _NCHUNK

    out_t = pl.pallas_call(
        _fused_mlp_kernel,
        in_specs=[
            pl.BlockSpec(memory_space=pl.ANY),
            pl.BlockSpec((hid, feat), lambda: (0, 0)),
            pl.BlockSpec((hid,), lambda: (0,)),
            pl.BlockSpec((hid,), lambda: (0,)),
            pl.BlockSpec((nclass, hid), lambda: (0, 0)),
            pl.BlockSpec((nclass,), lambda: (0,)),
        ],
        out_specs=pl.BlockSpec((nclass, n), lambda: (0, 0)),
        out_shape=jax.ShapeDtypeStruct((nclass, n), jnp.float32),
        scratch_shapes=[
            pltpu.VMEM((2, rows, feat), jnp.float32),
            pltpu.VMEM((n, hid), jnp.float32),
            pltpu.VMEM((8, hid), jnp.float32),
            pltpu.SemaphoreType.DMA((2,)),
        ],
    )(x, W1, gamma, beta, W2, b2)
    return out_t.T
